# Initial kernel scaffold; baseline (speedup 1.0000x reference)
import jax
import jax.numpy as jnp
from jax.experimental import pallas as pl


def kernel(node_rep, edge_rep, edge_index, We1, be1, We2, be2, Wn1, bn1, Wn2, bn2):
    raise NotImplementedError("write your pallas kernel here")


# speedup vs baseline: 3.2484x; 3.2484x over previous
import functools

import jax
import jax.numpy as jnp
from jax import lax
from jax.experimental import pallas as pl
from jax.experimental.pallas import tpu as pltpu
from jax.experimental.pallas import tpu_sc as plsc

N_NODES = 10000
N_EDGES = 320000
D = 128

EDGE_BLOCK = 2560
NODE_BLOCK = 2000

NUM_SC = 2
NUM_SUB = 16
NW = NUM_SC * NUM_SUB
E_PER_W = N_EDGES // NW
CHUNK = 80
N_CHUNKS = E_PER_W // CHUNK
N_PER_SUB = 624
N_LAST_SUB = N_NODES - (NUM_SUB - 1) * N_PER_SUB

_sc_mesh = plsc.VectorSubcoreMesh(core_axis_name="c", subcore_axis_name="s")


def _gather_body(node_hbm, src_hbm, dst_hbm, gs_hbm, gd_hbm,
                 idx_s, idx_d, rows_s, rows_d, sem_s, sem_d):
    wid = lax.axis_index("c") * NUM_SUB + lax.axis_index("s")
    base0 = wid * E_PER_W

    @pl.loop(0, N_CHUNKS)
    def _(k):
        base = base0 + k * CHUNK
        pltpu.sync_copy(src_hbm.at[pl.ds(base, CHUNK)], idx_s)
        pltpu.sync_copy(dst_hbm.at[pl.ds(base, CHUNK)], idx_d)
        cp_s = pltpu.async_copy(node_hbm.at[idx_s], rows_s, sem_s)
        cp_d = pltpu.async_copy(node_hbm.at[idx_d], rows_d, sem_d)
        cp_s.wait()
        pltpu.sync_copy(rows_s, gs_hbm.at[pl.ds(base, CHUNK)])
        cp_d.wait()
        pltpu.sync_copy(rows_d, gd_hbm.at[pl.ds(base, CHUNK)])


def _sc_gather(node_rep, src, dst):
    fn = pl.kernel(
        _gather_body,
        out_type=(jax.ShapeDtypeStruct((N_EDGES, D), jnp.float32),
                  jax.ShapeDtypeStruct((N_EDGES, D), jnp.float32)),
        mesh=_sc_mesh,
        scratch_types=[
            pltpu.VMEM((CHUNK,), jnp.int32),
            pltpu.VMEM((CHUNK,), jnp.int32),
            pltpu.VMEM((CHUNK, D), jnp.float32),
            pltpu.VMEM((CHUNK, D), jnp.float32),
            pltpu.SemaphoreType.DMA,
            pltpu.SemaphoreType.DMA,
        ],
    )
    return fn(node_rep, src, dst)


def _scatter_body(edge_out_hbm, src_hbm, dst_hbm, zeros_hbm, part_hbm,
                  idx_s, idx_d, rows, acc, sem):
    c = lax.axis_index("c")
    s = lax.axis_index("s")
    wid = c * NUM_SUB + s

    @pl.when(s < NUM_SUB - 1)
    def _():
        pltpu.sync_copy(zeros_hbm.at[pl.ds(s * N_PER_SUB, N_PER_SUB)],
                        acc.at[pl.ds(s * N_PER_SUB, N_PER_SUB)])

    @pl.when(s == NUM_SUB - 1)
    def _():
        pltpu.sync_copy(zeros_hbm.at[pl.ds(s * N_PER_SUB, N_LAST_SUB)],
                        acc.at[pl.ds(s * N_PER_SUB, N_LAST_SUB)])

    plsc.subcore_barrier()

    base0 = wid * E_PER_W

    @pl.loop(0, N_CHUNKS)
    def _(k):
        base = base0 + k * CHUNK
        pltpu.sync_copy(src_hbm.at[pl.ds(base, CHUNK)], idx_s)
        pltpu.sync_copy(dst_hbm.at[pl.ds(base, CHUNK)], idx_d)
        pltpu.async_copy(edge_out_hbm.at[pl.ds(base, CHUNK)], rows, sem).wait()
        pltpu.sync_copy(rows, acc.at[idx_s], add=True)
        pltpu.sync_copy(rows, acc.at[idx_d], add=True)

    plsc.subcore_barrier()

    @pl.when(s < NUM_SUB - 1)
    def _():
        pltpu.sync_copy(acc.at[pl.ds(s * N_PER_SUB, N_PER_SUB)],
                        part_hbm.at[c].at[pl.ds(s * N_PER_SUB, N_PER_SUB)])

    @pl.when(s == NUM_SUB - 1)
    def _():
        pltpu.sync_copy(acc.at[pl.ds(s * N_PER_SUB, N_LAST_SUB)],
                        part_hbm.at[c].at[pl.ds(s * N_PER_SUB, N_LAST_SUB)])


def _sc_scatter(edge_out, src, dst, zeros):
    fn = pl.kernel(
        _scatter_body,
        out_type=jax.ShapeDtypeStruct((NUM_SC, N_NODES, D), jnp.float32),
        mesh=_sc_mesh,
        scratch_types=[
            pltpu.VMEM((CHUNK,), jnp.int32),
            pltpu.VMEM((CHUNK,), jnp.int32),
            pltpu.VMEM((CHUNK, D), jnp.float32),
            pltpu.VMEM_SHARED((N_NODES, D), jnp.float32),
            pltpu.SemaphoreType.DMA,
        ],
    )
    return fn(edge_out, src, dst, zeros)


def _mlp_body(a_ref, b0_ref, b1_ref, w1_ref, b1b_ref, w2_ref, b2b_ref, out_ref):
    extra = b0_ref[...] + b1_ref[...]
    x = jnp.concatenate([a_ref[...], extra], axis=-1).astype(jnp.bfloat16)
    h = jnp.dot(x, w1_ref[...].astype(jnp.bfloat16),
                preferred_element_type=jnp.float32)
    h = jnp.maximum(h + b1b_ref[...], 0.0).astype(jnp.bfloat16)
    o = jnp.dot(h, w2_ref[...].astype(jnp.bfloat16),
                preferred_element_type=jnp.float32)
    out_ref[...] = jnp.maximum(o + b2b_ref[...], 0.0)


def _tc_mlp(a, b0, b1, W1, bias1, W2, bias2, block):
    n = a.shape[0]
    row = lambda i: (i, 0)
    full = lambda i: (0, 0)
    return pl.pallas_call(
        _mlp_body,
        grid=(n // block,),
        in_specs=[
            pl.BlockSpec((block, D), row),
            pl.BlockSpec((block, D), row),
            pl.BlockSpec((block, D), row),
            pl.BlockSpec((2 * D, 2 * D), full),
            pl.BlockSpec((1, 2 * D), full),
            pl.BlockSpec((2 * D, D), full),
            pl.BlockSpec((1, D), full),
        ],
        out_specs=pl.BlockSpec((block, D), row),
        out_shape=jax.ShapeDtypeStruct((n, D), jnp.float32),
    )(a, b0, b1, W1, bias1.reshape(1, -1), W2, bias2.reshape(1, -1))


def kernel(node_rep, edge_rep, edge_index, We1, be1, We2, be2, Wn1, bn1, Wn2, bn2):
    src = edge_index[0]
    dst = edge_index[1]
    gsrc, gdst = _sc_gather(node_rep, src, dst)
    edge_out = _tc_mlp(edge_rep, gsrc, gdst, We1, be1, We2, be2, EDGE_BLOCK)
    zeros = jnp.zeros((N_NODES, D), jnp.float32)
    partials = _sc_scatter(edge_out, src, dst, zeros)
    node_out = _tc_mlp(node_rep, partials[0], partials[1], Wn1, bn1, Wn2, bn2,
                       NODE_BLOCK)
    return (node_out, edge_out)
